# baseline (speedup 1.0000x reference)
import jax
import jax.numpy as jnp
from jax import lax
from jax.experimental import pallas as pl
from jax.experimental.pallas import tpu as pltpu, tpu_sc as plsc

N = 10000
NP = 10240
E = 320000
IN_DIM = 128
HID = 128
NCLS = 64
H0 = 2
NEG_SLOPE = 0.2

EPB = 128
EB = 2560
E_PAD = EB * EPB
BPT0 = EB // 16
BPT1 = EB // 32
RPT = NP // 16
SB = 16
BR = 256
EPS = 1e-9


def _tc_prep0(x, W0, al0, ar0):
    def body(x_ref, w_ref, al_ref, ar_ref, feat_ref, el_ref, er_ref):
        feat = jnp.dot(x_ref[...], w_ref[...], preferred_element_type=jnp.float32)
        fh = feat.reshape(BR, H0, HID).transpose(1, 0, 2)
        el_ref[...] = jnp.sum(fh * al_ref[...][:, None, :], axis=-1)
        er_ref[...] = jnp.sum(fh * ar_ref[...][:, None, :], axis=-1)
        feat_ref[...] = fh

    return pl.pallas_call(
        body,
        grid=(NP // BR,),
        in_specs=[
            pl.BlockSpec((BR, IN_DIM), lambda i: (i, 0)),
            pl.BlockSpec((IN_DIM, H0 * HID), lambda i: (0, 0)),
            pl.BlockSpec((H0, HID), lambda i: (0, 0)),
            pl.BlockSpec((H0, HID), lambda i: (0, 0)),
        ],
        out_specs=[
            pl.BlockSpec((H0, BR, HID), lambda i: (0, i, 0)),
            pl.BlockSpec((H0, BR), lambda i: (0, i)),
            pl.BlockSpec((H0, BR), lambda i: (0, i)),
        ],
        out_shape=[
            jax.ShapeDtypeStruct((H0, NP, HID), jnp.float32),
            jax.ShapeDtypeStruct((H0, NP), jnp.float32),
            jax.ShapeDtypeStruct((H0, NP), jnp.float32),
        ],
    )(x, W0, al0, ar0)


def _tc_mid(acc0, den0, b0, W1, al1, ar1):
    def body(acc_ref, den_ref, b_ref, w_ref, al_ref, ar_ref,
             f_ref, el_ref, er_ref):
        den = den_ref[...]
        hv = acc_ref[...] / (den[:, :, None] + EPS) + b_ref[...][:, None, :]
        hv = jnp.where(hv > 0, hv, jnp.exp(hv) - 1.0)
        hflat = hv.transpose(1, 0, 2).reshape(BR, H0 * HID)
        f1 = jnp.dot(hflat, w_ref[...], preferred_element_type=jnp.float32)
        el_ref[...] = jnp.sum(f1 * al_ref[...], axis=-1)
        er_ref[...] = jnp.sum(f1 * ar_ref[...], axis=-1)
        f_ref[...] = jnp.concatenate(
            [f1, jnp.zeros((BR, HID - NCLS), jnp.float32)], axis=1)

    return pl.pallas_call(
        body,
        grid=(NP // BR,),
        in_specs=[
            pl.BlockSpec((H0, BR, HID), lambda i: (0, i, 0)),
            pl.BlockSpec((H0, BR), lambda i: (0, i)),
            pl.BlockSpec((H0, HID), lambda i: (0, 0)),
            pl.BlockSpec((H0 * HID, NCLS), lambda i: (0, 0)),
            pl.BlockSpec((1, NCLS), lambda i: (0, 0)),
            pl.BlockSpec((1, NCLS), lambda i: (0, 0)),
        ],
        out_specs=[
            pl.BlockSpec((BR, HID), lambda i: (i, 0)),
            pl.BlockSpec((BR,), lambda i: (i,)),
            pl.BlockSpec((BR,), lambda i: (i,)),
        ],
        out_shape=[
            jax.ShapeDtypeStruct((NP, HID), jnp.float32),
            jax.ShapeDtypeStruct((NP,), jnp.float32),
            jax.ShapeDtypeStruct((NP,), jnp.float32),
        ],
    )(acc0, den0, b0, W1, al1, ar1)


def _tc_final(acc1, den1, b1):
    def body(acc_ref, den_ref, b_ref, out_ref):
        acc = acc_ref[0, :, :NCLS] + acc_ref[1, :, :NCLS]
        den = den_ref[0] + den_ref[1]
        o = acc / (den[:, None] + EPS) + b_ref[...][None, :]
        out_ref[...] = jnp.where(o > 0, o, jnp.exp(o) - 1.0)

    return pl.pallas_call(
        body,
        grid=(NP // BR,),
        in_specs=[
            pl.BlockSpec((2, BR, HID), lambda i: (0, i, 0)),
            pl.BlockSpec((2, BR), lambda i: (0, i)),
            pl.BlockSpec((NCLS,), lambda i: (0,)),
        ],
        out_specs=pl.BlockSpec((BR, NCLS), lambda i: (i, 0)),
        out_shape=jax.ShapeDtypeStruct((NP, NCLS), jnp.float32),
    )(acc1, den1, b1)


def _zero_2d(ref, rows, cols):
    def zb(i, _):
        ref[i // (cols // 16), pl.ds((i % (cols // 16)) * 16, 16)] = (
            jnp.zeros((16,), jnp.float32))
        return 0
    lax.fori_loop(0, rows * cols // 16, zb, 0)


def _zero_1d(ref, n):
    def zb(i, _):
        ref[pl.ds(i * 16, 16)] = jnp.zeros((16,), jnp.float32)
        return 0
    lax.fori_loop(0, n // 16, zb, 0)


def _score_body(bpt, head_split):
    def body(src_hbm, dst_hbm, el_hbm, er_hbm,
             s_hbm, g_hbm, den_hbm,
             el_v, er_v, src_v, dst_v, s_sb, g_sb, zb_v, den_s):
        c = lax.axis_index("c")
        s = lax.axis_index("s")
        _zero_1d(zb_v, EPB)
        for k in range(RPT // EPB):
            pltpu.sync_copy(zb_v, den_s.at[pl.ds(s * RPT + k * EPB, EPB)])
        if head_split:
            pltpu.sync_copy(el_hbm.at[pl.ds(c * NP, NP)], el_v)
            pltpu.sync_copy(er_hbm.at[pl.ds(c * NP, NP)], er_v)
            base = s * bpt
            off = c * NP
            orow = c * EB + base
        else:
            pltpu.sync_copy(el_hbm, el_v)
            pltpu.sync_copy(er_hbm, er_v)
            base = (c * 16 + s) * bpt
            off = 0
            orow = base
        plsc.subcore_barrier()

        def outer(sb, _):
            pltpu.sync_copy(src_hbm.at[pl.ds(base + sb * SB, SB)], src_v)
            pltpu.sync_copy(dst_hbm.at[pl.ds(base + sb * SB, SB)], dst_v)

            def blk(b, _):
                def grp(j, _):
                    s16 = src_v[b, pl.ds(j * 16, 16)]
                    d16 = dst_v[b, pl.ds(j * 16, 16)]
                    ev = (plsc.load_gather(el_v, [s16])
                          + plsc.load_gather(er_v, [d16]))
                    ev = jnp.where(ev > 0, ev, NEG_SLOPE * ev)
                    s_sb[b, pl.ds(j * 16, 16)] = jnp.exp(ev)
                    g_sb[b, pl.ds(j * 16, 16)] = s16 + off
                    return 0

                lax.fori_loop(0, EPB // 16, grp, 0, unroll=True)
                pltpu.sync_copy(s_sb.at[b], den_s.at[dst_v.at[b]], add=True)
                return 0

            lax.fori_loop(0, SB, blk, 0)
            pltpu.sync_copy(s_sb, s_hbm.at[pl.ds(orow + sb * SB, SB)])
            pltpu.sync_copy(g_sb, g_hbm.at[pl.ds(orow + sb * SB, SB)])
            return 0

        lax.fori_loop(0, bpt // SB, outer, 0)
        plsc.subcore_barrier()
        pltpu.sync_copy(den_s.at[pl.ds(s * RPT, RPT)],
                        den_hbm.at[pl.ds(c * NP + s * RPT, RPT)])

    return body


def _sc_score(src2d, dst2d, el, er, *, bpt, head_split):
    mesh = plsc.VectorSubcoreMesh(core_axis_name="c", subcore_axis_name="s",
                                  num_cores=2, num_subcores=16)
    f32 = jnp.float32
    nrow = 2 * EB if head_split else EB
    kfn = pl.kernel(
        _score_body(bpt, head_split),
        out_type=(jax.ShapeDtypeStruct((nrow, EPB), f32),
                  jax.ShapeDtypeStruct((nrow, EPB), jnp.int32),
                  jax.ShapeDtypeStruct((2 * NP,), f32)),
        mesh=mesh,
        compiler_params=pltpu.CompilerParams(needs_layout_passes=False),
        scratch_types=[
            pltpu.VMEM((NP,), f32),
            pltpu.VMEM((NP,), f32),
            pltpu.VMEM((SB, EPB), jnp.int32),
            pltpu.VMEM((SB, EPB), jnp.int32),
            pltpu.VMEM((SB, EPB), f32),
            pltpu.VMEM((SB, EPB), jnp.int32),
            pltpu.VMEM((EPB,), f32),
            pltpu.VMEM_SHARED((NP,), f32),
        ],
    )
    return kfn(src2d, dst2d, el, er)


def _spmm_body(bpt, head_split):
    def body(s_hbm, g_hbm, dst_hbm, feat_hbm, acc_hbm,
             s_sb, g_sb, dst_v, rows0, rows1, acc_s,
             sem0, sem1, csem0, csem1):
        c = lax.axis_index("c")
        s = lax.axis_index("s")
        _zero_2d(rows0, EPB, HID)
        for k in range(RPT // EPB):
            pltpu.sync_copy(rows0, acc_s.at[pl.ds(s * RPT + k * EPB, EPB)])
        if head_split:
            base = s * bpt
            orow = c * EB + base
        else:
            base = (c * 16 + s) * bpt
            orow = base
        plsc.subcore_barrier()

        def scale(rows_v, b):
            b16 = jnp.zeros((16,), jnp.int32) + b

            def edge(j, _):
                sj = plsc.load_gather(
                    s_sb, [b16, jnp.zeros((16,), jnp.int32) + j])
                for k in range(HID // 16):
                    rows_v[j, pl.ds(k * 16, 16)] = (
                        rows_v[j, pl.ds(k * 16, 16)] * sj)
                return 0

            lax.fori_loop(0, EPB, edge, 0, unroll=8)

        def outer(sb, _):
            pltpu.sync_copy(s_hbm.at[pl.ds(orow + sb * SB, SB)], s_sb)
            pltpu.sync_copy(g_hbm.at[pl.ds(orow + sb * SB, SB)], g_sb)
            pltpu.sync_copy(dst_hbm.at[pl.ds(base + sb * SB, SB)], dst_v)
            pltpu.async_copy(feat_hbm.at[g_sb.at[0]], rows0, sem0)

            def pair(k, _):
                b0 = 2 * k
                pltpu.make_async_copy(
                    feat_hbm.at[g_sb.at[b0]], rows0, sem0).wait()

                @pl.when(k > 0)
                def _():
                    pltpu.make_async_copy(
                        rows1, acc_s.at[dst_v.at[b0 - 1]], csem1).wait()

                pltpu.async_copy(feat_hbm.at[g_sb.at[b0 + 1]], rows1, sem1)
                scale(rows0, b0)
                pltpu.async_copy(
                    rows0, acc_s.at[dst_v.at[b0]], csem0, add=True)
                pltpu.make_async_copy(
                    feat_hbm.at[g_sb.at[b0 + 1]], rows1, sem1).wait()
                scale(rows1, b0 + 1)
                pltpu.async_copy(
                    rows1, acc_s.at[dst_v.at[b0 + 1]], csem1, add=True)

                @pl.when(k < SB // 2 - 1)
                def _():
                    pltpu.make_async_copy(
                        rows0, acc_s.at[dst_v.at[b0]], csem0).wait()
                    pltpu.async_copy(
                        feat_hbm.at[g_sb.at[b0 + 2]], rows0, sem0)

                return 0

            lax.fori_loop(0, SB // 2, pair, 0)
            pltpu.make_async_copy(
                rows0, acc_s.at[dst_v.at[SB - 2]], csem0).wait()
            pltpu.make_async_copy(
                rows1, acc_s.at[dst_v.at[SB - 1]], csem1).wait()
            return 0

        lax.fori_loop(0, bpt // SB, outer, 0)
        plsc.subcore_barrier()
        for k in range(RPT // EPB):
            r0 = s * RPT + k * EPB
            pltpu.sync_copy(acc_s.at[pl.ds(r0, EPB)],
                            acc_hbm.at[c].at[pl.ds(r0, EPB)])

    return body


def _sc_spmm(sflat, gflat, dst2d, feat, *, bpt, head_split):
    mesh = plsc.VectorSubcoreMesh(core_axis_name="c", subcore_axis_name="s",
                                  num_cores=2, num_subcores=16)
    f32 = jnp.float32
    kfn = pl.kernel(
        _spmm_body(bpt, head_split),
        out_type=jax.ShapeDtypeStruct((2, NP, HID), f32),
        mesh=mesh,
        compiler_params=pltpu.CompilerParams(needs_layout_passes=False),
        scratch_types=[
            pltpu.VMEM((SB, EPB), f32),
            pltpu.VMEM((SB, EPB), jnp.int32),
            pltpu.VMEM((SB, EPB), jnp.int32),
            pltpu.VMEM((EPB, HID), f32),
            pltpu.VMEM((EPB, HID), f32),
            pltpu.VMEM_SHARED((NP, HID), f32),
            pltpu.SemaphoreType.DMA,
            pltpu.SemaphoreType.DMA,
            pltpu.SemaphoreType.DMA,
            pltpu.SemaphoreType.DMA,
        ],
    )
    return kfn(sflat, gflat, dst2d, feat)


def kernel(inputs, edge_index, W0, attn_l0, attn_r0, b0,
           W1, attn_l1, attn_r1, b1):
    x = jnp.zeros((NP, IN_DIM), jnp.float32).at[:N].set(inputs)
    src = edge_index[0].astype(jnp.int32)
    dst = edge_index[1].astype(jnp.int32)
    pad = jnp.full((E_PAD - E,), N, jnp.int32)
    src2d = jnp.concatenate([src, pad]).reshape(EB, EPB)
    dst2d = jnp.concatenate([dst, pad]).reshape(EB, EPB)

    feat0, el0, er0 = _tc_prep0(x, W0, attn_l0, attn_r0)
    s0, g0, den0 = _sc_score(src2d, dst2d,
                             el0.reshape(H0 * NP), er0.reshape(H0 * NP),
                             bpt=BPT0, head_split=True)
    acc0 = _sc_spmm(s0, g0, dst2d, feat0.reshape(H0 * NP, HID),
                    bpt=BPT0, head_split=True)
    feat1, el1, er1 = _tc_mid(acc0, den0.reshape(H0, NP), b0.reshape(H0, HID),
                              W1, attn_l1, attn_r1)
    s1, g1, den1 = _sc_score(src2d, dst2d, el1, er1,
                             bpt=BPT1, head_split=False)
    acc1 = _sc_spmm(s1, g1, dst2d, feat1, bpt=BPT1, head_split=False)
    logits = _tc_final(acc1, den1.reshape(2, NP), b1)
    return logits[:N]


# speedup vs baseline: 1.1131x; 1.1131x over previous
import jax
import jax.numpy as jnp
from jax import lax
from jax.experimental import pallas as pl
from jax.experimental.pallas import tpu as pltpu, tpu_sc as plsc

N = 10000
NP = 10240
E = 320000
IN_DIM = 128
HID = 128
NCLS = 64
H0 = 2
NEG_SLOPE = 0.2

EPB = 128
EB = 2560
E_PAD = EB * EPB
BPT0 = EB // 16
BPT1 = EB // 32
RPT = NP // 16
SB = 16
BR = 256
EPS = 1e-9


def _tc_prep0(x, W0, al0, ar0):
    def body(x_ref, w_ref, al_ref, ar_ref, feat_ref, el_ref, er_ref):
        feat = jnp.dot(x_ref[...], w_ref[...], preferred_element_type=jnp.float32)
        fh = feat.reshape(BR, H0, HID).transpose(1, 0, 2)
        el_ref[...] = jnp.sum(fh * al_ref[...][:, None, :], axis=-1)
        er_ref[...] = jnp.sum(fh * ar_ref[...][:, None, :], axis=-1)
        feat_ref[...] = fh

    return pl.pallas_call(
        body,
        grid=(NP // BR,),
        in_specs=[
            pl.BlockSpec((BR, IN_DIM), lambda i: (i, 0)),
            pl.BlockSpec((IN_DIM, H0 * HID), lambda i: (0, 0)),
            pl.BlockSpec((H0, HID), lambda i: (0, 0)),
            pl.BlockSpec((H0, HID), lambda i: (0, 0)),
        ],
        out_specs=[
            pl.BlockSpec((H0, BR, HID), lambda i: (0, i, 0)),
            pl.BlockSpec((H0, BR), lambda i: (0, i)),
            pl.BlockSpec((H0, BR), lambda i: (0, i)),
        ],
        out_shape=[
            jax.ShapeDtypeStruct((H0, NP, HID), jnp.float32),
            jax.ShapeDtypeStruct((H0, NP), jnp.float32),
            jax.ShapeDtypeStruct((H0, NP), jnp.float32),
        ],
    )(x, W0, al0, ar0)


def _tc_mid(acc0, den0, b0, W1, al1, ar1):
    def body(acc_ref, den_ref, b_ref, w_ref, al_ref, ar_ref,
             f_ref, el_ref, er_ref):
        den = den_ref[...]
        hv = acc_ref[...] / (den[:, :, None] + EPS) + b_ref[...][:, None, :]
        hv = jnp.where(hv > 0, hv, jnp.exp(hv) - 1.0)
        hflat = hv.transpose(1, 0, 2).reshape(BR, H0 * HID)
        f1 = jnp.dot(hflat, w_ref[...], preferred_element_type=jnp.float32)
        el_ref[...] = jnp.sum(f1 * al_ref[...], axis=-1)
        er_ref[...] = jnp.sum(f1 * ar_ref[...], axis=-1)
        f_ref[...] = jnp.concatenate(
            [f1, jnp.zeros((BR, HID - NCLS), jnp.float32)], axis=1)

    return pl.pallas_call(
        body,
        grid=(NP // BR,),
        in_specs=[
            pl.BlockSpec((H0, BR, HID), lambda i: (0, i, 0)),
            pl.BlockSpec((H0, BR), lambda i: (0, i)),
            pl.BlockSpec((H0, HID), lambda i: (0, 0)),
            pl.BlockSpec((H0 * HID, NCLS), lambda i: (0, 0)),
            pl.BlockSpec((1, NCLS), lambda i: (0, 0)),
            pl.BlockSpec((1, NCLS), lambda i: (0, 0)),
        ],
        out_specs=[
            pl.BlockSpec((BR, HID), lambda i: (i, 0)),
            pl.BlockSpec((BR,), lambda i: (i,)),
            pl.BlockSpec((BR,), lambda i: (i,)),
        ],
        out_shape=[
            jax.ShapeDtypeStruct((NP, HID), jnp.float32),
            jax.ShapeDtypeStruct((NP,), jnp.float32),
            jax.ShapeDtypeStruct((NP,), jnp.float32),
        ],
    )(acc0, den0, b0, W1, al1, ar1)


def _tc_final(acc1, den1, b1):
    def body(acc_ref, den_ref, b_ref, out_ref):
        acc = acc_ref[0, :, :NCLS] + acc_ref[1, :, :NCLS]
        den = den_ref[0] + den_ref[1]
        o = acc / (den[:, None] + EPS) + b_ref[...][None, :]
        out_ref[...] = jnp.where(o > 0, o, jnp.exp(o) - 1.0)

    return pl.pallas_call(
        body,
        grid=(NP // BR,),
        in_specs=[
            pl.BlockSpec((2, BR, HID), lambda i: (0, i, 0)),
            pl.BlockSpec((2, BR), lambda i: (0, i)),
            pl.BlockSpec((NCLS,), lambda i: (0,)),
        ],
        out_specs=pl.BlockSpec((BR, NCLS), lambda i: (i, 0)),
        out_shape=jax.ShapeDtypeStruct((NP, NCLS), jnp.float32),
    )(acc1, den1, b1)


def _zero_2d(ref, rows, cols):
    def zb(i, _):
        ref[i // (cols // 16), pl.ds((i % (cols // 16)) * 16, 16)] = (
            jnp.zeros((16,), jnp.float32))
        return 0
    lax.fori_loop(0, rows * cols // 16, zb, 0)


def _zero_1d(ref, n):
    def zb(i, _):
        ref[pl.ds(i * 16, 16)] = jnp.zeros((16,), jnp.float32)
        return 0
    lax.fori_loop(0, n // 16, zb, 0)


def _score_body(bpt, head_split):
    def body(src_hbm, dst_hbm, el_hbm, er_hbm,
             s_hbm, g_hbm, den_hbm,
             el_v, er_v, src_v, dst_v, s_sb, g_sb, zb_v, den_s):
        c = lax.axis_index("c")
        s = lax.axis_index("s")
        _zero_1d(zb_v, EPB)
        for k in range(RPT // EPB):
            pltpu.sync_copy(zb_v, den_s.at[pl.ds(s * RPT + k * EPB, EPB)])
        if head_split:
            pltpu.sync_copy(el_hbm.at[pl.ds(c * NP, NP)], el_v)
            pltpu.sync_copy(er_hbm.at[pl.ds(c * NP, NP)], er_v)
            base = s * bpt
            off = c * NP
            orow = c * EB + base
        else:
            pltpu.sync_copy(el_hbm, el_v)
            pltpu.sync_copy(er_hbm, er_v)
            base = (c * 16 + s) * bpt
            off = 0
            orow = base
        plsc.subcore_barrier()

        def outer(sb, _):
            pltpu.sync_copy(src_hbm.at[pl.ds(base + sb * SB, SB)], src_v)
            pltpu.sync_copy(dst_hbm.at[pl.ds(base + sb * SB, SB)], dst_v)

            def blk(b, _):
                def grp(j, _):
                    s16 = src_v[b, pl.ds(j * 16, 16)]
                    d16 = dst_v[b, pl.ds(j * 16, 16)]
                    ev = (plsc.load_gather(el_v, [s16])
                          + plsc.load_gather(er_v, [d16]))
                    ev = jnp.where(ev > 0, ev, NEG_SLOPE * ev)
                    s_sb[b, pl.ds(j * 16, 16)] = jnp.exp(ev)
                    g_sb[b, pl.ds(j * 16, 16)] = s16 + off
                    return 0

                lax.fori_loop(0, EPB // 16, grp, 0, unroll=True)
                pltpu.sync_copy(s_sb.at[b], den_s.at[dst_v.at[b]], add=True)
                return 0

            lax.fori_loop(0, SB, blk, 0)
            pltpu.sync_copy(s_sb, s_hbm.at[pl.ds(orow + sb * SB, SB)])
            pltpu.sync_copy(g_sb, g_hbm.at[pl.ds(orow + sb * SB, SB)])
            return 0

        lax.fori_loop(0, bpt // SB, outer, 0)
        plsc.subcore_barrier()
        pltpu.sync_copy(den_s.at[pl.ds(s * RPT, RPT)],
                        den_hbm.at[pl.ds(c * NP + s * RPT, RPT)])

    return body


def _sc_score(src2d, dst2d, el, er, *, bpt, head_split):
    mesh = plsc.VectorSubcoreMesh(core_axis_name="c", subcore_axis_name="s",
                                  num_cores=2, num_subcores=16)
    f32 = jnp.float32
    nrow = 2 * EB if head_split else EB
    kfn = pl.kernel(
        _score_body(bpt, head_split),
        out_type=(jax.ShapeDtypeStruct((nrow, EPB), f32),
                  jax.ShapeDtypeStruct((nrow, EPB), jnp.int32),
                  jax.ShapeDtypeStruct((2 * NP,), f32)),
        mesh=mesh,
        compiler_params=pltpu.CompilerParams(needs_layout_passes=False),
        scratch_types=[
            pltpu.VMEM((NP,), f32),
            pltpu.VMEM((NP,), f32),
            pltpu.VMEM((SB, EPB), jnp.int32),
            pltpu.VMEM((SB, EPB), jnp.int32),
            pltpu.VMEM((SB, EPB), f32),
            pltpu.VMEM((SB, EPB), jnp.int32),
            pltpu.VMEM((EPB,), f32),
            pltpu.VMEM_SHARED((NP,), f32),
        ],
    )
    return kfn(src2d, dst2d, el, er)


def _spmm_body(bpt, head_split):
    def body(s_hbm, g_hbm, dst_hbm, feat_hbm, acc_hbm,
             s_sb, g_sb, dst_v, rows0, rows1, acc_s,
             sem0, sem1, csem0, csem1):
        c = lax.axis_index("c")
        s = lax.axis_index("s")
        _zero_2d(rows0, EPB, HID)
        for k in range(RPT // EPB):
            pltpu.sync_copy(rows0, acc_s.at[pl.ds(s * RPT + k * EPB, EPB)])
        if head_split:
            base = s * bpt
            orow = c * EB + base
        else:
            base = (c * 16 + s) * bpt
            orow = base
        plsc.subcore_barrier()

        def scale(rows_v, b):
            b16 = jnp.zeros((16,), jnp.int32) + b

            def edge(j):
                sj = plsc.load_gather(
                    s_sb, [b16, jnp.zeros((16,), jnp.int32) + j])
                for k in range(HID // 16):
                    rows_v[j, pl.ds(k * 16, 16)] = (
                        rows_v[j, pl.ds(k * 16, 16)] * sj)

            plsc.parallel_loop(0, EPB, unroll=8)(edge)

        def outer(sb, _):
            pltpu.sync_copy(s_hbm.at[pl.ds(orow + sb * SB, SB)], s_sb)
            pltpu.sync_copy(g_hbm.at[pl.ds(orow + sb * SB, SB)], g_sb)
            pltpu.sync_copy(dst_hbm.at[pl.ds(base + sb * SB, SB)], dst_v)
            pltpu.async_copy(feat_hbm.at[g_sb.at[0]], rows0, sem0)

            def pair(k, _):
                b0 = 2 * k
                pltpu.async_copy(feat_hbm.at[g_sb.at[b0 + 1]], rows1, sem1)
                pltpu.make_async_copy(
                    feat_hbm.at[g_sb.at[b0]], rows0, sem0).wait()
                scale(rows0, b0)
                pltpu.sync_copy(rows0, acc_s.at[dst_v.at[b0]], add=True)

                @pl.when(k < SB // 2 - 1)
                def _():
                    pltpu.async_copy(
                        feat_hbm.at[g_sb.at[b0 + 2]], rows0, sem0)

                pltpu.make_async_copy(
                    feat_hbm.at[g_sb.at[b0 + 1]], rows1, sem1).wait()
                scale(rows1, b0 + 1)
                pltpu.sync_copy(rows1, acc_s.at[dst_v.at[b0 + 1]], add=True)
                return 0

            lax.fori_loop(0, SB // 2, pair, 0)
            return 0

        lax.fori_loop(0, bpt // SB, outer, 0)
        plsc.subcore_barrier()
        for k in range(RPT // EPB):
            r0 = s * RPT + k * EPB
            pltpu.sync_copy(acc_s.at[pl.ds(r0, EPB)],
                            acc_hbm.at[c].at[pl.ds(r0, EPB)])

    return body


def _sc_spmm(sflat, gflat, dst2d, feat, *, bpt, head_split):
    mesh = plsc.VectorSubcoreMesh(core_axis_name="c", subcore_axis_name="s",
                                  num_cores=2, num_subcores=16)
    f32 = jnp.float32
    kfn = pl.kernel(
        _spmm_body(bpt, head_split),
        out_type=jax.ShapeDtypeStruct((2, NP, HID), f32),
        mesh=mesh,
        compiler_params=pltpu.CompilerParams(needs_layout_passes=False),
        scratch_types=[
            pltpu.VMEM((SB, EPB), f32),
            pltpu.VMEM((SB, EPB), jnp.int32),
            pltpu.VMEM((SB, EPB), jnp.int32),
            pltpu.VMEM((EPB, HID), f32),
            pltpu.VMEM((EPB, HID), f32),
            pltpu.VMEM_SHARED((NP, HID), f32),
            pltpu.SemaphoreType.DMA,
            pltpu.SemaphoreType.DMA,
            pltpu.SemaphoreType.DMA,
            pltpu.SemaphoreType.DMA,
        ],
    )
    return kfn(sflat, gflat, dst2d, feat)


def kernel(inputs, edge_index, W0, attn_l0, attn_r0, b0,
           W1, attn_l1, attn_r1, b1):
    x = jnp.zeros((NP, IN_DIM), jnp.float32).at[:N].set(inputs)
    src = edge_index[0].astype(jnp.int32)
    dst = edge_index[1].astype(jnp.int32)
    pad = jnp.full((E_PAD - E,), N, jnp.int32)
    src2d = jnp.concatenate([src, pad]).reshape(EB, EPB)
    dst2d = jnp.concatenate([dst, pad]).reshape(EB, EPB)

    feat0, el0, er0 = _tc_prep0(x, W0, attn_l0, attn_r0)
    s0, g0, den0 = _sc_score(src2d, dst2d,
                             el0.reshape(H0 * NP), er0.reshape(H0 * NP),
                             bpt=BPT0, head_split=True)
    acc0 = _sc_spmm(s0, g0, dst2d, feat0.reshape(H0 * NP, HID),
                    bpt=BPT0, head_split=True)
    feat1, el1, er1 = _tc_mid(acc0, den0.reshape(H0, NP), b0.reshape(H0, HID),
                              W1, attn_l1, attn_r1)
    s1, g1, den1 = _sc_score(src2d, dst2d, el1, er1,
                             bpt=BPT1, head_split=False)
    acc1 = _sc_spmm(s1, g1, dst2d, feat1, bpt=BPT1, head_split=False)
    logits = _tc_final(acc1, den1.reshape(2, NP), b1)
    return logits[:N]
